# baseline (speedup 1.0000x reference)
import functools

import jax
import jax.numpy as jnp
from jax import lax
from jax.experimental import pallas as pl
from jax.experimental.pallas import tpu as pltpu
from jax.experimental.pallas import tpu_sc as plsc

NR, NE, E = 4, 10000, 640000
DIN = DOUT = 128
NB = 2

K = 100
NCHUNK = E // K
NSUB = 16
CPS = NCHUNK // NSUB
IB = 40
NBLK = CPS // IB
ZROWS = 24
RPS = 624
REM = NE - NSUB * RPS


NBUF = 2


def _sc_body(h_hbm, esrc_hbm, edst_hbm, g_hbm,
             g_sp, zbuf, isrc, idst, rows, sems):
    c = lax.axis_index("c")
    s = lax.axis_index("s")

    def _zb(i, carry):
        zbuf[i // 8, pl.ds((i % 8) * 16, 16)] = jnp.zeros((16,), jnp.float32)
        return carry
    lax.fori_loop(0, ZROWS * 8, _zb, 0)

    for rr in range(2):
        r = c * 2 + rr

        for kz in range(RPS // ZROWS):
            pltpu.sync_copy(zbuf, g_sp.at[pl.ds(kz * ZROWS, ZROWS)])

        plsc.subcore_barrier()

        c0 = s * CPS
        for b in range(NBLK):
            base = c0 + b * IB
            pltpu.sync_copy(esrc_hbm.at[r, pl.ds(base, IB)], isrc)
            pltpu.sync_copy(edst_hbm.at[r, pl.ds(base, IB)], idst)
            for u in range(NBUF):
                pltpu.async_copy(h_hbm.at[pl.ds(u * 192, 192)], rows[u], sems[u])

            def _ring(q, carry):
                for u in range(NBUF):
                    j = q * NBUF + u
                    pltpu.make_async_copy(h_hbm.at[pl.ds(0, 192)],
                                          rows[u], sems[u]).wait()

                    @pl.when(j + NBUF < IB)
                    def _():
                        pltpu.async_copy(h_hbm.at[pl.ds((j % 32) * 192, 192)],
                                         rows[u], sems[u])
                return carry

            lax.fori_loop(0, IB // NBUF, _ring, 0)

        plsc.subcore_barrier()
        pltpu.sync_copy(g_sp.at[pl.ds(0, RPS)],
                        g_hbm.at[r, pl.ds(s * RPS, RPS)])

        plsc.subcore_barrier()


@functools.lru_cache(maxsize=1)
def _sc_segsum():
    return pl.kernel(
        _sc_body,
        out_type=jax.ShapeDtypeStruct((NR, NE, DIN), jnp.float32),
        mesh=plsc.VectorSubcoreMesh(core_axis_name="c", subcore_axis_name="s",
                                    num_cores=2, num_subcores=NSUB),
        scratch_types=[
            pltpu.VMEM_SHARED((RPS + REM, DIN), jnp.float32),
            pltpu.VMEM((ZROWS, DIN), jnp.float32),
            pltpu.VMEM((IB, K), jnp.int32),
            pltpu.VMEM((IB, K), jnp.int32),
            [pltpu.VMEM((192, DIN), jnp.float32) for _ in range(NBUF)],
            [pltpu.SemaphoreType.DMA for _ in range(NBUF)],
        ],
    )


BLK = 1000


def _mm_body(a_ref, g_ref, v_ref, o_ref):
    v0 = v_ref[0]
    v1 = v_ref[1]
    acc = jnp.zeros((BLK, DOUT), jnp.float32)
    for r in range(NR):
        w = a_ref[r, 0] * v0 + a_ref[r, 1] * v1
        acc = acc + jnp.dot(g_ref[r], w, preferred_element_type=jnp.float32)
    o_ref[...] = acc


def _mm(G, V, A):
    return pl.pallas_call(
        _mm_body,
        grid=(NE // BLK,),
        in_specs=[
            pl.BlockSpec(memory_space=pltpu.SMEM),
            pl.BlockSpec((NR, BLK, DIN), lambda i: (0, i, 0)),
            pl.BlockSpec((NB, DIN, DOUT), lambda i: (0, 0, 0)),
        ],
        out_specs=pl.BlockSpec((BLK, DOUT), lambda i: (i, 0)),
        out_shape=jax.ShapeDtypeStruct((NE, DOUT), jnp.float32),
    )(A, G, V)


def kernel(H, edge_index, V, A):
    esrc = edge_index[:, 1, :].reshape(NR, NCHUNK, K)
    edst = edge_index[:, 0, :].reshape(NR, NCHUNK, K)
    G = _sc_segsum()(H, esrc, edst)
    return _mm(G, V, A)


# speedup vs baseline: 1.5687x; 1.5687x over previous
import functools

import jax
import jax.numpy as jnp
from jax import lax
from jax.experimental import pallas as pl
from jax.experimental.pallas import tpu as pltpu
from jax.experimental.pallas import tpu_sc as plsc

NR, NE, E = 4, 10000, 640000
DIN = DOUT = 128
NB = 2

K = 128
NCHUNK = E // K
NSUB = 16
CPS = 312
TAIL = NCHUNK - NSUB * CPS
IB = 24
NBLK = CPS // IB
NBUF = 2
ZROWS = 24
RPS = 624
REM = NE - NSUB * RPS


def _sc_body(h_hbm, esrc_hbm, edst_hbm, g_hbm,
             g_sp, zbuf, isrc, idst, rows, sems):
    c = lax.axis_index("c")
    s = lax.axis_index("s")

    def _zb(i, carry):
        zbuf[i // 8, pl.ds((i % 8) * 16, 16)] = jnp.zeros((16,), jnp.float32)
        return carry
    lax.fori_loop(0, ZROWS * 8, _zb, 0)

    for rr in range(2):
        r = c * 2 + rr

        for kz in range(RPS // ZROWS):
            pltpu.sync_copy(zbuf, g_sp.at[pl.ds(s * RPS + kz * ZROWS, ZROWS)])

        @pl.when(s == NSUB - 1)
        def _():
            pltpu.sync_copy(zbuf.at[pl.ds(0, REM)],
                            g_sp.at[pl.ds(NSUB * RPS, REM)])

        plsc.subcore_barrier()

        c0 = s * CPS
        for b in range(NBLK):
            base = c0 + b * IB
            pltpu.sync_copy(esrc_hbm.at[r, pl.ds(base, IB)], isrc)
            pltpu.sync_copy(edst_hbm.at[r, pl.ds(base, IB)], idst)
            for u in range(NBUF):
                pltpu.async_copy(h_hbm.at[isrc.at[u]], rows[u], sems[u])

            def _ring(q, carry):
                for u in range(NBUF):
                    j = q * NBUF + u
                    pltpu.make_async_copy(h_hbm.at[isrc.at[j]],
                                          rows[u], sems[u]).wait()
                    pltpu.sync_copy(rows[u], g_sp.at[idst.at[j]], add=True)

                    @pl.when(j + NBUF < IB)
                    def _():
                        pltpu.async_copy(h_hbm.at[isrc.at[j + NBUF]],
                                         rows[u], sems[u])
                return carry

            lax.fori_loop(0, IB // NBUF, _ring, 0)

        @pl.when(s < TAIL)
        def _():
            pltpu.sync_copy(esrc_hbm.at[r, pl.ds(NSUB * CPS, TAIL)],
                            isrc.at[pl.ds(0, TAIL)])
            pltpu.sync_copy(edst_hbm.at[r, pl.ds(NSUB * CPS, TAIL)],
                            idst.at[pl.ds(0, TAIL)])
            pltpu.async_copy(h_hbm.at[isrc.at[s]], rows[0], sems[0])
            pltpu.make_async_copy(h_hbm.at[isrc.at[s]], rows[0], sems[0]).wait()
            pltpu.sync_copy(rows[0], g_sp.at[idst.at[s]], add=True)

        plsc.subcore_barrier()
        pltpu.sync_copy(g_sp.at[pl.ds(s * RPS, RPS)],
                        g_hbm.at[r, pl.ds(s * RPS, RPS)])

        @pl.when(s == NSUB - 1)
        def _():
            pltpu.sync_copy(g_sp.at[pl.ds(NSUB * RPS, REM)],
                            g_hbm.at[r, pl.ds(NSUB * RPS, REM)])

        plsc.subcore_barrier()


@functools.lru_cache(maxsize=1)
def _sc_segsum():
    return pl.kernel(
        _sc_body,
        out_type=jax.ShapeDtypeStruct((NR, NE, DIN), jnp.float32),
        mesh=plsc.VectorSubcoreMesh(core_axis_name="c", subcore_axis_name="s",
                                    num_cores=2, num_subcores=NSUB),
        scratch_types=[
            pltpu.VMEM_SHARED((NE, DIN), jnp.float32),
            pltpu.VMEM((ZROWS, DIN), jnp.float32),
            pltpu.VMEM((IB, K), jnp.int32),
            pltpu.VMEM((IB, K), jnp.int32),
            [pltpu.VMEM((K, DIN), jnp.float32) for _ in range(NBUF)],
            [pltpu.SemaphoreType.DMA for _ in range(NBUF)],
        ],
    )


BLK = 1000


def _mm_body(a_ref, g_ref, v_ref, o_ref):
    v0 = v_ref[0]
    v1 = v_ref[1]
    acc = jnp.zeros((BLK, DOUT), jnp.float32)
    for r in range(NR):
        w = a_ref[r, 0] * v0 + a_ref[r, 1] * v1
        acc = acc + jnp.dot(g_ref[r], w, preferred_element_type=jnp.float32)
    o_ref[...] = acc


def _mm(G, V, A):
    return pl.pallas_call(
        _mm_body,
        grid=(NE // BLK,),
        in_specs=[
            pl.BlockSpec(memory_space=pltpu.SMEM),
            pl.BlockSpec((NR, BLK, DIN), lambda i: (0, i, 0)),
            pl.BlockSpec((NB, DIN, DOUT), lambda i: (0, 0, 0)),
        ],
        out_specs=pl.BlockSpec((BLK, DOUT), lambda i: (i, 0)),
        out_shape=jax.ShapeDtypeStruct((NE, DOUT), jnp.float32),
    )(A, G, V)


def kernel(H, edge_index, V, A):
    esrc = edge_index[:, 1, :].reshape(NR, NCHUNK, K)
    edst = edge_index[:, 0, :].reshape(NR, NCHUNK, K)
    G = _sc_segsum()(H, esrc, edst)
    return _mm(G, V, A)


# speedup vs baseline: 1.6274x; 1.0374x over previous
import functools

import jax
import jax.numpy as jnp
from jax import lax
from jax.experimental import pallas as pl
from jax.experimental.pallas import tpu as pltpu
from jax.experimental.pallas import tpu_sc as plsc

NR, NE, E = 4, 10000, 640000
DIN = DOUT = 128
NB = 2

K = 128
NCHUNK = E // K
NSUB = 16
CPS = 312
TAIL = NCHUNK - NSUB * CPS
IB = 24
NBLK = CPS // IB
NBUF = 2
ZROWS = 24
RPS = 624
REM = NE - NSUB * RPS


def _sc_body(h_hbm, esrc_hbm, edst_hbm, g_hbm,
             g_sp, zbuf, isrc, idst, isems, dsems, rows, sems):
    c = lax.axis_index("c")
    s = lax.axis_index("s")

    def _zb(i, carry):
        zbuf[i // 8, pl.ds((i % 8) * 16, 16)] = jnp.zeros((16,), jnp.float32)
        return carry
    lax.fori_loop(0, ZROWS * 8, _zb, 0)

    for rr in range(2):
        r = c * 2 + rr

        for kz in range(RPS // ZROWS):
            pltpu.sync_copy(zbuf, g_sp.at[pl.ds(s * RPS + kz * ZROWS, ZROWS)])

        @pl.when(s == NSUB - 1)
        def _():
            pltpu.sync_copy(zbuf.at[pl.ds(0, REM)],
                            g_sp.at[pl.ds(NSUB * RPS, REM)])

        plsc.subcore_barrier()

        c0 = s * CPS
        pltpu.async_copy(esrc_hbm.at[r, pl.ds(c0, IB)], isrc[0], isems[0])
        pltpu.async_copy(edst_hbm.at[r, pl.ds(c0, IB)], idst[0], dsems[0])
        for b in range(NBLK):
            p = b % 2
            pltpu.make_async_copy(esrc_hbm.at[r, pl.ds(c0, IB)],
                                  isrc[p], isems[p]).wait()
            pltpu.make_async_copy(edst_hbm.at[r, pl.ds(c0, IB)],
                                  idst[p], dsems[p]).wait()
            if b + 1 < NBLK:
                nbase = c0 + (b + 1) * IB
                pltpu.async_copy(esrc_hbm.at[r, pl.ds(nbase, IB)],
                                 isrc[1 - p], isems[1 - p])
                pltpu.async_copy(edst_hbm.at[r, pl.ds(nbase, IB)],
                                 idst[1 - p], dsems[1 - p])
            for u in range(NBUF):
                pltpu.async_copy(h_hbm.at[isrc[p].at[u]], rows[u], sems[u])

            def _ring(q, carry):
                for u in range(NBUF):
                    j = q * NBUF + u
                    pltpu.make_async_copy(h_hbm.at[isrc[p].at[j]],
                                          rows[u], sems[u]).wait()
                    pltpu.sync_copy(rows[u], g_sp.at[idst[p].at[j]], add=True)

                    @pl.when(j + NBUF < IB)
                    def _():
                        pltpu.async_copy(h_hbm.at[isrc[p].at[j + NBUF]],
                                         rows[u], sems[u])
                return carry

            lax.fori_loop(0, IB // NBUF, _ring, 0)

        @pl.when(s < TAIL)
        def _():
            pltpu.sync_copy(esrc_hbm.at[r, pl.ds(NSUB * CPS, TAIL)],
                            isrc[NBLK % 2].at[pl.ds(0, TAIL)])
            pltpu.sync_copy(edst_hbm.at[r, pl.ds(NSUB * CPS, TAIL)],
                            idst[NBLK % 2].at[pl.ds(0, TAIL)])
            pltpu.async_copy(h_hbm.at[isrc[NBLK % 2].at[s]], rows[0], sems[0])
            pltpu.make_async_copy(h_hbm.at[isrc[NBLK % 2].at[s]],
                                  rows[0], sems[0]).wait()
            pltpu.sync_copy(rows[0], g_sp.at[idst[NBLK % 2].at[s]], add=True)

        plsc.subcore_barrier()
        pltpu.sync_copy(g_sp.at[pl.ds(s * RPS, RPS)],
                        g_hbm.at[r, pl.ds(s * RPS, RPS)])

        @pl.when(s == NSUB - 1)
        def _():
            pltpu.sync_copy(g_sp.at[pl.ds(NSUB * RPS, REM)],
                            g_hbm.at[r, pl.ds(NSUB * RPS, REM)])

        plsc.subcore_barrier()


@functools.lru_cache(maxsize=1)
def _sc_segsum():
    return pl.kernel(
        _sc_body,
        out_type=jax.ShapeDtypeStruct((NR, NE, DIN), jnp.float32),
        mesh=plsc.VectorSubcoreMesh(core_axis_name="c", subcore_axis_name="s",
                                    num_cores=2, num_subcores=NSUB),
        scratch_types=[
            pltpu.VMEM_SHARED((NE, DIN), jnp.float32),
            pltpu.VMEM((ZROWS, DIN), jnp.float32),
            [pltpu.VMEM((IB, K), jnp.int32) for _ in range(2)],
            [pltpu.VMEM((IB, K), jnp.int32) for _ in range(2)],
            [pltpu.SemaphoreType.DMA for _ in range(2)],
            [pltpu.SemaphoreType.DMA for _ in range(2)],
            [pltpu.VMEM((K, DIN), jnp.float32) for _ in range(NBUF)],
            [pltpu.SemaphoreType.DMA for _ in range(NBUF)],
        ],
    )


BLK = 1000


def _mm_body(a_ref, g_ref, v_ref, o_ref):
    v0 = v_ref[0]
    v1 = v_ref[1]
    acc = jnp.zeros((BLK, DOUT), jnp.float32)
    for r in range(NR):
        w = a_ref[r, 0] * v0 + a_ref[r, 1] * v1
        acc = acc + jnp.dot(g_ref[r], w, preferred_element_type=jnp.float32)
    o_ref[...] = acc


def _mm(G, V, A):
    return pl.pallas_call(
        _mm_body,
        grid=(NE // BLK,),
        in_specs=[
            pl.BlockSpec(memory_space=pltpu.SMEM),
            pl.BlockSpec((NR, BLK, DIN), lambda i: (0, i, 0)),
            pl.BlockSpec((NB, DIN, DOUT), lambda i: (0, 0, 0)),
        ],
        out_specs=pl.BlockSpec((BLK, DOUT), lambda i: (i, 0)),
        out_shape=jax.ShapeDtypeStruct((NE, DOUT), jnp.float32),
    )(A, G, V)


def kernel(H, edge_index, V, A):
    esrc = edge_index[:, 1, :].reshape(NR, NCHUNK, K)
    edst = edge_index[:, 0, :].reshape(NR, NCHUNK, K)
    G = _sc_segsum()(H, esrc, edst)
    return _mm(G, V, A)
